# baseline (speedup 1.0000x reference)
import functools

import jax
import jax.numpy as jnp
from jax import lax
from jax.experimental import pallas as pl
from jax.experimental.pallas import tpu as pltpu
from jax.experimental.pallas import tpu_sc as plsc

N = 10000
NP = 10240
NR = NP // 128
HALF = NP // 2
D = 128
E = 320000
TILES = 16
G = 128
GRP = 8
NCH = 160
EPT = NCH * G
EP = TILES * EPT
NSL = HALF // TILES
BLK = 1280


def _tc_pre_body(x_ref, w_ref, a_ref, h_ref, al_ref):
    h = jnp.dot(x_ref[...], w_ref[0], preferred_element_type=jnp.float32)
    h_ref[0] = h
    al_ref[0, 0] = jnp.sum(h * a_ref[0, 0], axis=1)
    al_ref[0, 1] = jnp.sum(h * a_ref[0, 1], axis=1)


def _tc_pre(xp, W01, att):
    return pl.pallas_call(
        _tc_pre_body,
        grid=(2, NP // BLK),
        in_specs=[
            pl.BlockSpec((BLK, D), lambda c, i: (i, 0)),
            pl.BlockSpec((1, D, D), lambda c, i: (c, 0, 0)),
            pl.BlockSpec((1, 8, D), lambda c, i: (c, 0, 0)),
        ],
        out_specs=[
            pl.BlockSpec((1, BLK, D), lambda c, i: (c, i, 0)),
            pl.BlockSpec((1, 2, BLK), lambda c, i: (c, 0, i)),
        ],
        out_shape=[
            jax.ShapeDtypeStruct((2, NP, D), jnp.float32),
            jax.ShapeDtypeStruct((2, 2, NP), jnp.float32),
        ],
    )(xp, W01, att)


def _sc_body(src_hbm, dst_hbm, h_hbm, alpha_hbm,
             raw_o, sc_o,
             srcg_v, dstg_v, idxh_v, idxd_v, rows_v, rows2_v,
             asrc_v, adst_v, acc1_v, acc2_v,
             e_g, idxe_v, idxa_v, idx_v, idxw_v, idxsc_v,
             raw_sh, red_sh, sem, semg0, semg1, sems0, sems1):
    c = lax.axis_index("c")
    sid = lax.axis_index("s")
    zero16 = jnp.zeros((16,), jnp.float32)
    one16 = jnp.ones((16,), jnp.float32)
    cnp16 = jnp.full((16,), c * NP, jnp.int32)
    iota16 = lax.iota(jnp.int32, 16)

    for i in range(NCH // 16):
        idxe_v[pl.ds(i * 16, 16)] = iota16 + (sid * NCH + 16 * i)
    for i in range(NR // 16):
        idxa_v[0, pl.ds(i * 16, 16)] = iota16 + (c * 2 * NR + 16 * i)
        idxa_v[2, pl.ds(i * 16, 16)] = iota16 + (c * 2 * NR + NR + 16 * i)
        idx_v[pl.ds(i * 16, 16)] = iota16 + (16 * i)
    for k in range(3):
        for i in range(NR // 16):
            idxsc_v[2 * k, pl.ds(i * 16, 16)] = iota16 + (
                c * 3 * NR + k * NR + 16 * i)

    pltpu.async_copy(alpha_hbm.at[idxa_v.at[0]], asrc_v, sem).wait()
    pltpu.async_copy(alpha_hbm.at[idxa_v.at[2]], adst_v, sem).wait()

    def z_body(r, carry):
        for l in range(D // 16):
            sl = pl.ds(l * 16, 16)
            acc1_v[r, sl] = zero16
            acc2_v[r, sl] = zero16
        return carry
    lax.fori_loop(0, NR, z_body, 0)

    def zrows_body(r, carry):
        for l in range(D // 16):
            rows_v[r, pl.ds(l * 16, 16)] = zero16
        return carry
    lax.fori_loop(0, G, zrows_body, 0)

    def zero_raw_sh():
        pltpu.sync_copy(rows_v, raw_sh.at[pl.ds(sid * NSL, G)])
        pltpu.sync_copy(rows_v, raw_sh.at[pl.ds(sid * NSL + G, G)])
        pltpu.sync_copy(rows_v.at[pl.ds(0, NSL - 2 * G)],
                        raw_sh.at[pl.ds(sid * NSL + 2 * G, NSL - 2 * G)])
        pltpu.sync_copy(rows_v.at[pl.ds(0, 8)],
                        raw_sh.at[pl.ds(HALF + sid * 8, 8)])
    zero_raw_sh()
    @pl.when(sid < NR // 8)
    def _():
        pltpu.sync_copy(rows_v.at[pl.ds(0, 8)], red_sh.at[pl.ds(sid * 8, 8)])
    plsc.subcore_barrier()

    def split(i16):
        return [lax.shift_right_logical(i16, 7), lax.bitwise_and(i16, 127)]

    def stage_group(g, hp_base):
        gsl = pl.ds(0, GRP)
        del gsl
        pltpu.async_copy(src_hbm.at[idxe_v.at[pl.ds(g * GRP, GRP)]],
                         srcg_v, sem).wait()
        pltpu.async_copy(dst_hbm.at[idxe_v.at[pl.ds(g * GRP, GRP)]],
                         dstg_v, sem).wait()
        def gb(r, carry):
            for l in range(D // 16):
                sl = pl.ds(l * 16, 16)
                s16 = srcg_v[r, sl]
                d16 = dstg_v[r, sl]
                dr = d16 - hp_base
                inr = jnp.logical_and(dr >= 0, dr < HALF)
                idxh_v[r, sl] = jnp.where(inr, s16 + cnp16, cnp16)
                dump = (jnp.full((16,), HALF + 8 * sid, jnp.int32)
                        + lax.bitwise_and(d16, 7))
                idxd_v[r, sl] = jnp.where(inr, dr, dump)
            return carry
        lax.fori_loop(0, GRP, gb, 0)

    def escalar8(do_acc):
        for m in range(GRP):
            for k in range(G // 16):
                ksl = pl.ds(k * 16, 16)
                s16 = srcg_v[m, ksl]
                d16 = dstg_v[m, ksl]
                a = (plsc.load_gather(asrc_v, split(s16))
                     + plsc.load_gather(adst_v, split(d16)))
                a = jnp.where(a >= 0, a, 0.2 * a)
                ev = jnp.exp(a)
                e_g[pl.ds(m * G + k * 16, 16)] = ev
                if do_acc:
                    plsc.addupdate_scatter(acc1_v, split(d16), ev)
                    plsc.addupdate_scatter(acc2_v, split(s16), one16)

    def scale_rows(buf, m):
        def row_body(rr, rc):
            r = 2 * rr
            ev_a = plsc.load_gather(
                e_g, [jnp.full((16,), m * G, jnp.int32) + r])
            ev_b = plsc.load_gather(
                e_g, [jnp.full((16,), m * G + 1, jnp.int32) + r])
            for l in range(D // 16):
                rl = pl.ds(l * 16, 16)
                buf[r, rl] = buf[r, rl] * ev_a
                buf[r + 1, rl] = buf[r + 1, rl] * ev_b
            return rc
        lax.fori_loop(0, G // 2, row_body, 0)

    bufs = (rows_v, rows2_v)
    gsems = (semg0, semg1)
    ssems = (sems0, sems1)

    for hp in range(2):
        hp_base = hp * HALF

        def group_body(g, carry, hp_base=hp_base, do_acc=(hp == 0)):
            stage_group(g, hp_base)
            gd = [None] * GRP
            sd = [None] * GRP
            gd[0] = pltpu.async_copy(
                h_hbm.at[idxh_v.at[0]], bufs[0], gsems[0])
            escalar8(do_acc)
            for m in range(GRP):
                b = m % 2
                gd[m].wait()
                if m >= 2:
                    sd[m - 2].wait()
                if m + 1 < GRP:
                    gd[m + 1] = pltpu.async_copy(
                        h_hbm.at[idxh_v.at[m + 1]], bufs[1 - b],
                        gsems[1 - b])
                scale_rows(bufs[b], m)
                sd[m] = pltpu.async_copy(
                    bufs[b], raw_sh.at[idxd_v.at[m]], ssems[b], add=True)
            sd[GRP - 2].wait()
            sd[GRP - 1].wait()
            return carry
        lax.fori_loop(0, NCH // GRP, group_body, 0)
        plsc.subcore_barrier()

        for q in range(NSL // 64):
            for m in range(64 // 16):
                idxw_v[0, pl.ds(m * 16, 16)] = iota16 + (
                    c * NP + hp_base + sid * NSL + q * 64 + m * 16)
            pltpu.sync_copy(raw_sh.at[pl.ds(sid * NSL + q * 64, 64)],
                            rows_v.at[pl.ds(0, 64)])
            pltpu.async_copy(rows_v.at[pl.ds(0, 64)],
                             raw_o.at[idxw_v.at[0]], sem).wait()
        if hp == 0:
            def zrows2(r, carry):
                for l in range(D // 16):
                    rows_v[r, pl.ds(l * 16, 16)] = zero16
                return carry
            lax.fori_loop(0, G, zrows2, 0)
            zero_raw_sh()
            plsc.subcore_barrier()

    pltpu.sync_copy(acc1_v, red_sh.at[idx_v], add=True)
    plsc.subcore_barrier()
    pltpu.sync_copy(red_sh, acc1_v)
    plsc.subcore_barrier()
    @pl.when(sid == 0)
    def _():
        pltpu.async_copy(acc1_v, sc_o.at[idxsc_v.at[0]], sem).wait()
    def zr8(r, carry):
        for l in range(D // 16):
            rows_v[r, pl.ds(l * 16, 16)] = zero16
        return carry
    lax.fori_loop(0, 8, zr8, 0)
    @pl.when(sid < NR // 8)
    def _():
        pltpu.sync_copy(rows_v.at[pl.ds(0, 8)], red_sh.at[pl.ds(sid * 8, 8)])
    plsc.subcore_barrier()
    pltpu.sync_copy(acc2_v, red_sh.at[idx_v], add=True)
    plsc.subcore_barrier()
    @pl.when(sid == 1)
    def _():
        pltpu.sync_copy(red_sh, acc2_v)
        pltpu.async_copy(acc2_v, sc_o.at[idxsc_v.at[2]], sem).wait()
    plsc.subcore_barrier()
    def za2(r, carry):
        for l in range(D // 16):
            acc2_v[r, pl.ds(l * 16, 16)] = zero16
        return carry
    lax.fori_loop(0, NR, za2, 0)
    @pl.when(sid < NR // 8)
    def _():
        pltpu.sync_copy(rows_v.at[pl.ds(0, 8)], red_sh.at[pl.ds(sid * 8, 8)])
    plsc.subcore_barrier()

    def p2_body(j, carry):
        g = lax.shift_right_logical(j, 3)
        jj = lax.bitwise_and(j, 7)
        @pl.when(jj == 0)
        def _():
            pltpu.async_copy(src_hbm.at[idxe_v.at[pl.ds(g * GRP, GRP)]],
                             srcg_v, sem).wait()
            pltpu.async_copy(dst_hbm.at[idxe_v.at[pl.ds(g * GRP, GRP)]],
                             dstg_v, sem).wait()
        for k in range(G // 16):
            ksl = pl.ds(k * 16, 16)
            s16 = srcg_v[jj, ksl]
            d16 = dstg_v[jj, ksl]
            a = (plsc.load_gather(asrc_v, split(s16))
                 + plsc.load_gather(adst_v, split(d16)))
            a = jnp.where(a >= 0, a, 0.2 * a)
            ev = jnp.exp(a)
            dn = plsc.load_gather(acc1_v, split(d16))
            attn = ev / (dn + 1e-16)
            plsc.addupdate_scatter(acc2_v, split(s16), attn)
        return carry
    lax.fori_loop(0, NCH, p2_body, 0)
    pltpu.sync_copy(acc2_v, red_sh.at[idx_v], add=True)
    plsc.subcore_barrier()
    @pl.when(sid == 2)
    def _():
        pltpu.sync_copy(red_sh, acc2_v)
        pltpu.async_copy(acc2_v, sc_o.at[idxsc_v.at[4]], sem).wait()


def _sc_edge(src, dst, hflat, alpha):
    mesh = plsc.VectorSubcoreMesh(core_axis_name="c", subcore_axis_name="s")
    fn = functools.partial(
        pl.kernel,
        out_type=[
            jax.ShapeDtypeStruct((2 * NP, D), jnp.float32),
            jax.ShapeDtypeStruct((2 * 3 * NR, 128), jnp.float32),
        ],
        mesh=mesh,
        scratch_types=[
            pltpu.VMEM((GRP, G), jnp.int32),
            pltpu.VMEM((GRP, G), jnp.int32),
            pltpu.VMEM((GRP, G), jnp.int32),
            pltpu.VMEM((GRP, G), jnp.int32),
            pltpu.VMEM((G, D), jnp.float32),
            pltpu.VMEM((G, D), jnp.float32),
            pltpu.VMEM((NR, 128), jnp.float32),
            pltpu.VMEM((NR, 128), jnp.float32),
            pltpu.VMEM((NR, 128), jnp.float32),
            pltpu.VMEM((NR, 128), jnp.float32),
            pltpu.VMEM((GRP * G,), jnp.float32),
            pltpu.VMEM((NCH,), jnp.int32),
            pltpu.VMEM((4, NR), jnp.int32),
            pltpu.VMEM((NR,), jnp.int32),
            pltpu.VMEM((2, 64), jnp.int32),
            pltpu.VMEM((6, NR), jnp.int32),
            pltpu.VMEM_SHARED((HALF + 8 * TILES, D), jnp.float32),
            pltpu.VMEM_SHARED((NR, 128), jnp.float32),
            pltpu.SemaphoreType.DMA,
            pltpu.SemaphoreType.DMA,
            pltpu.SemaphoreType.DMA,
            pltpu.SemaphoreType.DMA,
            pltpu.SemaphoreType.DMA,
        ],
        compiler_params=pltpu.CompilerParams(needs_layout_passes=False),
    )(_sc_body)
    return fn(src, dst, hflat, alpha)


def _tc_post_body(r_ref, d_ref, s_ref, c_ref, f_ref, n_ref):
    eps = jnp.float32(1e-16)
    f0 = r_ref[0] / (d_ref[0] + eps)
    f1 = r_ref[1] / (d_ref[1] + eps)
    f_ref[...] = (jnp.where(f0 >= 0, f0, 0.01 * f0)
                  + jnp.where(f1 >= 0, f1, 0.01 * f1))
    n_ref[...] = (s_ref[0] + s_ref[1]) / jnp.maximum(c_ref[0], 1.0)


def _tc_post(raw, den3, s3, cnt3):
    return pl.pallas_call(
        _tc_post_body,
        grid=(NP // BLK,),
        in_specs=[
            pl.BlockSpec((2, BLK, D), lambda i: (0, i, 0)),
            pl.BlockSpec((2, BLK, 1), lambda i: (0, i, 0)),
            pl.BlockSpec((2, BLK, 1), lambda i: (0, i, 0)),
            pl.BlockSpec((2, BLK, 1), lambda i: (0, i, 0)),
        ],
        out_specs=[
            pl.BlockSpec((BLK, D), lambda i: (i, 0)),
            pl.BlockSpec((BLK, 1), lambda i: (i, 0)),
        ],
        out_shape=[
            jax.ShapeDtypeStruct((NP, D), jnp.float32),
            jax.ShapeDtypeStruct((NP, 1), jnp.float32),
        ],
    )(raw, den3, s3, cnt3)


def kernel(x, edge_index, W0, att_src0, att_dst0, W1, att_src1, att_dst1):
    xp = jnp.zeros((NP, D), jnp.float32).at[:N].set(x)
    W01 = jnp.stack([W0, W1])
    att = jnp.zeros((2, 8, D), jnp.float32)
    att = (att.at[0, 0].set(att_src0).at[0, 1].set(att_dst0)
              .at[1, 0].set(att_src1).at[1, 1].set(att_dst1))
    h, alpha = _tc_pre(xp, W01, att)
    hflat = h.reshape(2 * NP, D)
    alpha2 = alpha.reshape(2 * 2 * NR, 128)

    pad = jnp.full((EP - E,), N, jnp.int32)
    src = jnp.concatenate([edge_index[0], pad]).reshape(TILES * NCH, G)
    dst = jnp.concatenate([edge_index[1], pad]).reshape(TILES * NCH, G)

    rawf, scal = _sc_edge(src, dst, hflat, alpha2)
    raw = rawf.reshape(2, NP, D)
    scal = scal.reshape(2, 3, NP)
    den3 = scal[:, 0][..., None]
    cnt3 = scal[:, 1][..., None]
    s3 = scal[:, 2][..., None]
    feat, ns = _tc_post(raw, den3, s3, cnt3)
    return feat[:N], ns[:N, 0]


# speedup vs baseline: 8.8398x; 8.8398x over previous
import functools

import jax
import jax.numpy as jnp
from jax import lax
from jax.experimental import pallas as pl
from jax.experimental.pallas import tpu as pltpu
from jax.experimental.pallas import tpu_sc as plsc

N = 10000
NP = 10240
NR = NP // 128
HALF = NP // 2
D = 128
E = 320000
TILES = 16
G = 128
GRP = 8
NCH = 160
EPT = NCH * G
EP = TILES * EPT
NSL = HALF // TILES
BLK = 1280


def _tc_pre_body(x_ref, w_ref, a_ref, h_ref, al_ref):
    h = jnp.dot(x_ref[...], w_ref[0], preferred_element_type=jnp.float32)
    h_ref[0] = h
    al_ref[0, 0] = jnp.sum(h * a_ref[0, 0], axis=1)
    al_ref[0, 1] = jnp.sum(h * a_ref[0, 1], axis=1)


def _tc_pre(xp, W01, att):
    return pl.pallas_call(
        _tc_pre_body,
        grid=(2, NP // BLK),
        in_specs=[
            pl.BlockSpec((BLK, D), lambda c, i: (i, 0)),
            pl.BlockSpec((1, D, D), lambda c, i: (c, 0, 0)),
            pl.BlockSpec((1, 8, D), lambda c, i: (c, 0, 0)),
        ],
        out_specs=[
            pl.BlockSpec((1, BLK, D), lambda c, i: (c, i, 0)),
            pl.BlockSpec((1, 2, BLK), lambda c, i: (c, 0, i)),
        ],
        out_shape=[
            jax.ShapeDtypeStruct((2, NP, D), jnp.float32),
            jax.ShapeDtypeStruct((2, 2, NP), jnp.float32),
        ],
    )(xp, W01, att)


def _sc_body(src_hbm, dst_hbm, h_hbm, alpha_hbm,
             raw_o, sc_o,
             srcg_v, dstg_v, idxh_v, idxd_v, rows_v, rows2_v,
             asrc_v, adst_v, acc1_v, acc2_v,
             e_g, idxe_v, idxa_v, idx_v, idxw_v, idxsc_v,
             raw_sh, red_sh, sem, semg0, semg1, sems0, sems1):
    c = lax.axis_index("c")
    sid = lax.axis_index("s")
    zero16 = jnp.zeros((16,), jnp.float32)
    one16 = jnp.ones((16,), jnp.float32)
    cnp16 = jnp.full((16,), c * NP, jnp.int32)
    iota16 = lax.iota(jnp.int32, 16)

    for i in range(NCH // 16):
        idxe_v[pl.ds(i * 16, 16)] = iota16 + (sid * NCH + 16 * i)
    for i in range(NR // 16):
        idxa_v[0, pl.ds(i * 16, 16)] = iota16 + (c * 2 * NR + 16 * i)
        idxa_v[2, pl.ds(i * 16, 16)] = iota16 + (c * 2 * NR + NR + 16 * i)
        idx_v[pl.ds(i * 16, 16)] = iota16 + (16 * i)
    for k in range(3):
        for i in range(NR // 16):
            idxsc_v[2 * k, pl.ds(i * 16, 16)] = iota16 + (
                c * 3 * NR + k * NR + 16 * i)

    pltpu.async_copy(alpha_hbm.at[idxa_v.at[0]], asrc_v, sem).wait()
    pltpu.async_copy(alpha_hbm.at[idxa_v.at[2]], adst_v, sem).wait()

    def z_body(r, carry):
        for l in range(D // 16):
            sl = pl.ds(l * 16, 16)
            acc1_v[r, sl] = zero16
            acc2_v[r, sl] = zero16
        return carry
    lax.fori_loop(0, NR, z_body, 0)

    def zrows_body(r, carry):
        for l in range(D // 16):
            rows_v[r, pl.ds(l * 16, 16)] = zero16
        return carry
    lax.fori_loop(0, G, zrows_body, 0)

    def zero_raw_sh():
        pltpu.sync_copy(rows_v, raw_sh.at[pl.ds(sid * NSL, G)])
        pltpu.sync_copy(rows_v, raw_sh.at[pl.ds(sid * NSL + G, G)])
        pltpu.sync_copy(rows_v.at[pl.ds(0, NSL - 2 * G)],
                        raw_sh.at[pl.ds(sid * NSL + 2 * G, NSL - 2 * G)])
        pltpu.sync_copy(rows_v.at[pl.ds(0, 8)],
                        raw_sh.at[pl.ds(HALF + sid * 8, 8)])
    zero_raw_sh()
    @pl.when(sid < NR // 8)
    def _():
        pltpu.sync_copy(rows_v.at[pl.ds(0, 8)], red_sh.at[pl.ds(sid * 8, 8)])
    plsc.subcore_barrier()

    def split(i16):
        return [lax.shift_right_logical(i16, 7), lax.bitwise_and(i16, 127)]

    def stage_group(g, hp_base):
        gsl = pl.ds(0, GRP)
        del gsl
        pltpu.async_copy(src_hbm.at[idxe_v.at[pl.ds(g * GRP, GRP)]],
                         srcg_v, sem).wait()
        pltpu.async_copy(dst_hbm.at[idxe_v.at[pl.ds(g * GRP, GRP)]],
                         dstg_v, sem).wait()
        def gb(r, carry):
            for l in range(D // 16):
                sl = pl.ds(l * 16, 16)
                s16 = srcg_v[r, sl]
                d16 = dstg_v[r, sl]
                dr = d16 - hp_base
                inr = jnp.logical_and(dr >= 0, dr < HALF)
                idxh_v[r, sl] = s16 + cnp16
                dump = (jnp.full((16,), HALF + 8 * sid, jnp.int32)
                        + lax.bitwise_and(d16, 7))
                idxd_v[r, sl] = jnp.where(inr, dr, dump)
            return carry
        lax.fori_loop(0, GRP, gb, 0)

    def escalar8(do_acc):
        for m in range(GRP):
            for k in range(G // 16):
                ksl = pl.ds(k * 16, 16)
                s16 = srcg_v[m, ksl]
                d16 = dstg_v[m, ksl]
                a = (plsc.load_gather(asrc_v, split(s16))
                     + plsc.load_gather(adst_v, split(d16)))
                a = jnp.where(a >= 0, a, 0.2 * a)
                ev = jnp.exp(a)
                e_g[pl.ds(m * G + k * 16, 16)] = ev
                if do_acc:
                    plsc.addupdate_scatter(acc1_v, split(d16), ev)
                    plsc.addupdate_scatter(acc2_v, split(s16), one16)

    def scale_rows(buf, m):
        def row_body(rr, rc):
            r = 2 * rr
            ev_a = plsc.load_gather(
                e_g, [jnp.full((16,), m * G, jnp.int32) + r])
            ev_b = plsc.load_gather(
                e_g, [jnp.full((16,), m * G + 1, jnp.int32) + r])
            for l in range(D // 16):
                rl = pl.ds(l * 16, 16)
                buf[r, rl] = buf[r, rl] * ev_a
                buf[r + 1, rl] = buf[r + 1, rl] * ev_b
            return rc
        lax.fori_loop(0, G // 2, row_body, 0)

    bufs = (rows_v, rows2_v)
    gsems = (semg0, semg1)
    ssems = (sems0, sems1)

    for hp in range(2):
        hp_base = hp * HALF

        def group_body(g, carry, hp_base=hp_base, do_acc=(hp == 0)):
            stage_group(g, hp_base)
            gd = [None] * GRP
            sd = [None] * GRP
            gd[0] = pltpu.async_copy(
                h_hbm.at[idxh_v.at[0]], bufs[0], gsems[0])
            escalar8(do_acc)
            for m in range(GRP):
                b = m % 2
                gd[m].wait()
                if m >= 2:
                    sd[m - 2].wait()
                if m + 1 < GRP:
                    gd[m + 1] = pltpu.async_copy(
                        h_hbm.at[idxh_v.at[m + 1]], bufs[1 - b],
                        gsems[1 - b])
                scale_rows(bufs[b], m)
                sd[m] = pltpu.async_copy(
                    bufs[b], raw_sh.at[idxd_v.at[m]], ssems[b], add=True)
            sd[GRP - 2].wait()
            sd[GRP - 1].wait()
            return carry
        lax.fori_loop(0, NCH // GRP, group_body, 0)
        plsc.subcore_barrier()

        for q in range(NSL // 64):
            for m in range(64 // 16):
                idxw_v[0, pl.ds(m * 16, 16)] = iota16 + (
                    c * NP + hp_base + sid * NSL + q * 64 + m * 16)
            pltpu.sync_copy(raw_sh.at[pl.ds(sid * NSL + q * 64, 64)],
                            rows_v.at[pl.ds(0, 64)])
            pltpu.async_copy(rows_v.at[pl.ds(0, 64)],
                             raw_o.at[idxw_v.at[0]], sem).wait()
        if hp == 0:
            def zrows2(r, carry):
                for l in range(D // 16):
                    rows_v[r, pl.ds(l * 16, 16)] = zero16
                return carry
            lax.fori_loop(0, G, zrows2, 0)
            zero_raw_sh()
            plsc.subcore_barrier()

    pltpu.sync_copy(acc1_v, red_sh.at[idx_v], add=True)
    plsc.subcore_barrier()
    pltpu.sync_copy(red_sh, acc1_v)
    plsc.subcore_barrier()
    @pl.when(sid == 0)
    def _():
        pltpu.async_copy(acc1_v, sc_o.at[idxsc_v.at[0]], sem).wait()
    def zr8(r, carry):
        for l in range(D // 16):
            rows_v[r, pl.ds(l * 16, 16)] = zero16
        return carry
    lax.fori_loop(0, 8, zr8, 0)
    @pl.when(sid < NR // 8)
    def _():
        pltpu.sync_copy(rows_v.at[pl.ds(0, 8)], red_sh.at[pl.ds(sid * 8, 8)])
    plsc.subcore_barrier()
    pltpu.sync_copy(acc2_v, red_sh.at[idx_v], add=True)
    plsc.subcore_barrier()
    @pl.when(sid == 1)
    def _():
        pltpu.sync_copy(red_sh, acc2_v)
        pltpu.async_copy(acc2_v, sc_o.at[idxsc_v.at[2]], sem).wait()
    plsc.subcore_barrier()
    def za2(r, carry):
        for l in range(D // 16):
            acc2_v[r, pl.ds(l * 16, 16)] = zero16
        return carry
    lax.fori_loop(0, NR, za2, 0)
    @pl.when(sid < NR // 8)
    def _():
        pltpu.sync_copy(rows_v.at[pl.ds(0, 8)], red_sh.at[pl.ds(sid * 8, 8)])
    plsc.subcore_barrier()

    def p2_body(j, carry):
        g = lax.shift_right_logical(j, 3)
        jj = lax.bitwise_and(j, 7)
        @pl.when(jj == 0)
        def _():
            pltpu.async_copy(src_hbm.at[idxe_v.at[pl.ds(g * GRP, GRP)]],
                             srcg_v, sem).wait()
            pltpu.async_copy(dst_hbm.at[idxe_v.at[pl.ds(g * GRP, GRP)]],
                             dstg_v, sem).wait()
        for k in range(G // 16):
            ksl = pl.ds(k * 16, 16)
            s16 = srcg_v[jj, ksl]
            d16 = dstg_v[jj, ksl]
            a = (plsc.load_gather(asrc_v, split(s16))
                 + plsc.load_gather(adst_v, split(d16)))
            a = jnp.where(a >= 0, a, 0.2 * a)
            ev = jnp.exp(a)
            dn = plsc.load_gather(acc1_v, split(d16))
            attn = ev / (dn + 1e-16)
            plsc.addupdate_scatter(acc2_v, split(s16), attn)
        return carry
    lax.fori_loop(0, NCH, p2_body, 0)
    pltpu.sync_copy(acc2_v, red_sh.at[idx_v], add=True)
    plsc.subcore_barrier()
    @pl.when(sid == 2)
    def _():
        pltpu.sync_copy(red_sh, acc2_v)
        pltpu.async_copy(acc2_v, sc_o.at[idxsc_v.at[4]], sem).wait()


def _sc_edge(src, dst, hflat, alpha):
    mesh = plsc.VectorSubcoreMesh(core_axis_name="c", subcore_axis_name="s")
    fn = functools.partial(
        pl.kernel,
        out_type=[
            jax.ShapeDtypeStruct((2 * NP, D), jnp.float32),
            jax.ShapeDtypeStruct((2 * 3 * NR, 128), jnp.float32),
        ],
        mesh=mesh,
        scratch_types=[
            pltpu.VMEM((GRP, G), jnp.int32),
            pltpu.VMEM((GRP, G), jnp.int32),
            pltpu.VMEM((GRP, G), jnp.int32),
            pltpu.VMEM((GRP, G), jnp.int32),
            pltpu.VMEM((G, D), jnp.float32),
            pltpu.VMEM((G, D), jnp.float32),
            pltpu.VMEM((NR, 128), jnp.float32),
            pltpu.VMEM((NR, 128), jnp.float32),
            pltpu.VMEM((NR, 128), jnp.float32),
            pltpu.VMEM((NR, 128), jnp.float32),
            pltpu.VMEM((GRP * G,), jnp.float32),
            pltpu.VMEM((NCH,), jnp.int32),
            pltpu.VMEM((4, NR), jnp.int32),
            pltpu.VMEM((NR,), jnp.int32),
            pltpu.VMEM((2, 64), jnp.int32),
            pltpu.VMEM((6, NR), jnp.int32),
            pltpu.VMEM_SHARED((HALF + 8 * TILES, D), jnp.float32),
            pltpu.VMEM_SHARED((NR, 128), jnp.float32),
            pltpu.SemaphoreType.DMA,
            pltpu.SemaphoreType.DMA,
            pltpu.SemaphoreType.DMA,
            pltpu.SemaphoreType.DMA,
            pltpu.SemaphoreType.DMA,
        ],
        compiler_params=pltpu.CompilerParams(needs_layout_passes=False),
    )(_sc_body)
    return fn(src, dst, hflat, alpha)


def _tc_post_body(r_ref, d_ref, s_ref, c_ref, f_ref, n_ref):
    eps = jnp.float32(1e-16)
    f0 = r_ref[0] / (d_ref[0] + eps)
    f1 = r_ref[1] / (d_ref[1] + eps)
    f_ref[...] = (jnp.where(f0 >= 0, f0, 0.01 * f0)
                  + jnp.where(f1 >= 0, f1, 0.01 * f1))
    n_ref[...] = (s_ref[0] + s_ref[1]) / jnp.maximum(c_ref[0], 1.0)


def _tc_post(raw, den3, s3, cnt3):
    return pl.pallas_call(
        _tc_post_body,
        grid=(NP // BLK,),
        in_specs=[
            pl.BlockSpec((2, BLK, D), lambda i: (0, i, 0)),
            pl.BlockSpec((2, BLK, 1), lambda i: (0, i, 0)),
            pl.BlockSpec((2, BLK, 1), lambda i: (0, i, 0)),
            pl.BlockSpec((2, BLK, 1), lambda i: (0, i, 0)),
        ],
        out_specs=[
            pl.BlockSpec((BLK, D), lambda i: (i, 0)),
            pl.BlockSpec((BLK, 1), lambda i: (i, 0)),
        ],
        out_shape=[
            jax.ShapeDtypeStruct((NP, D), jnp.float32),
            jax.ShapeDtypeStruct((NP, 1), jnp.float32),
        ],
    )(raw, den3, s3, cnt3)


def kernel(x, edge_index, W0, att_src0, att_dst0, W1, att_src1, att_dst1):
    xp = jnp.zeros((NP, D), jnp.float32).at[:N].set(x)
    W01 = jnp.stack([W0, W1])
    att = jnp.zeros((2, 8, D), jnp.float32)
    att = (att.at[0, 0].set(att_src0).at[0, 1].set(att_dst0)
              .at[1, 0].set(att_src1).at[1, 1].set(att_dst1))
    h, alpha = _tc_pre(xp, W01, att)
    hflat = h.reshape(2 * NP, D)
    alpha2 = alpha.reshape(2 * 2 * NR, 128)

    pad = jnp.full((EP - E,), N, jnp.int32)
    src = jnp.concatenate([edge_index[0], pad]).reshape(TILES * NCH, G)
    dst = jnp.concatenate([edge_index[1], pad]).reshape(TILES * NCH, G)

    rawf, scal = _sc_edge(src, dst, hflat, alpha2)
    raw = rawf.reshape(2, NP, D)
    scal = scal.reshape(2, 3, NP)
    den3 = scal[:, 0][..., None]
    cnt3 = scal[:, 1][..., None]
    s3 = scal[:, 2][..., None]
    feat, ns = _tc_post(raw, den3, s3, cnt3)
    return feat[:N], ns[:N, 0]
